# baseline (speedup 1.0000x reference)
import functools

import jax
import jax.numpy as jnp
from jax import lax
from jax.experimental import pallas as pl
from jax.experimental.pallas import tpu as pltpu
from jax.experimental.pallas import tpu_sc as plsc

_EPS = 1e-5
_PACK = 8
_BE = 3200

_NC = 2
_NS = 16
_NW = _NC * _NS
_C = 1024
_G = _C // 16


_LOG2E = 1.4426950408889634
_LN2 = 0.6931471805599453


def _softplus(z):
    return jnp.maximum(z, 0.0) + _LN2 * jnp.log2(1.0 + jnp.exp2(jnp.abs(z) * -_LOG2E))


def _make_stats_body(nl):
    def body(*args):
        x_ref = args[0]
        p = 1
        h = x_ref[...]
        for _ in range(nl):
            wb = args[p][...]
            a = args[p + 1][...]
            c2 = args[p + 2][...]
            p += 3
            h = _softplus(jnp.dot(h, wb, preferred_element_type=jnp.float32) * a + c2)
        wb = args[p][...]
        sum_ref = args[p + 1]
        sq_ref = args[p + 2]
        h = jnp.dot(h, wb, preferred_element_type=jnp.float32)

        @pl.when(pl.program_id(0) == 0)
        def _():
            sum_ref[...] = jnp.zeros_like(sum_ref)
            sq_ref[...] = jnp.zeros_like(sq_ref)

        sum_ref[...] += jnp.sum(h, axis=0, keepdims=True)
        sq_ref[...] += jnp.sum(h * h, axis=0, keepdims=True)

    return body


def _make_final_body(nl):
    def body(*args):
        x_ref = args[0]
        p = 1
        h = x_ref[...]
        for _ in range(nl):
            wb = args[p][...]
            a = args[p + 1][...]
            c2 = args[p + 2][...]
            p += 3
            h = _softplus(jnp.dot(h, wb, preferred_element_type=jnp.float32) * a + c2)
        wo = args[p][...]
        bo = args[p + 1]
        out_ref = args[p + 2]
        s8 = jnp.dot(h, wo, preferred_element_type=jnp.float32) + bo[0, 0]
        out_ref[...] = s8.T

    return body


def _full_spec():
    return pl.BlockSpec((128, 128), lambda i: (0, 0))


def _vec_spec():
    return pl.BlockSpec((1, 128), lambda i: (0, 0))


def _stats_pass(xp, known, wb_next, grid):
    ops = [xp]
    specs = [pl.BlockSpec((_BE, 128), lambda i: (i, 0))]
    for (wb, a, c2) in known:
        ops += [wb, a, c2]
        specs += [_full_spec(), _vec_spec(), _vec_spec()]
    ops += [wb_next]
    specs += [_full_spec()]
    return pl.pallas_call(
        _make_stats_body(len(known)),
        grid=(grid,),
        in_specs=specs,
        out_specs=[_vec_spec(), _vec_spec()],
        out_shape=[jax.ShapeDtypeStruct((1, 128), jnp.float32)] * 2,
    )(*ops)


def _final_pass(xp, known, wo_sel, bo_t, grid, rows):
    ops = [xp]
    specs = [pl.BlockSpec((_BE, 128), lambda i: (i, 0))]
    for (wb, a, c2) in known:
        ops += [wb, a, c2]
        specs += [_full_spec(), _vec_spec(), _vec_spec()]
    ops += [wo_sel, bo_t]
    specs += [pl.BlockSpec((128, _PACK), lambda i: (0, 0)), _vec_spec()]
    return pl.pallas_call(
        _make_final_body(len(known)),
        grid=(grid,),
        in_specs=specs,
        out_specs=pl.BlockSpec((_PACK, _BE), lambda i: (0, i)),
        out_shape=jax.ShapeDtypeStruct((_PACK, rows), jnp.float32),
    )(*ops)


def _fold_stats(sum_v, sq_v, count, b, g, be):
    m_raw = sum_v.reshape(_PACK, 16).sum(axis=0) / count
    q_raw = sq_v.reshape(_PACK, 16).sum(axis=0) / count
    mean = m_raw + b
    var = q_raw + 2.0 * b * m_raw + b * b - mean * mean
    a16 = g / jnp.sqrt(var + _EPS)
    c16 = be - mean * a16
    c2 = b * a16 + c16
    return jnp.tile(a16, _PACK)[None, :], jnp.tile(c2, _PACK)[None, :]


def _newton_rsqrt(x):
    i = lax.bitcast_convert_type(x, jnp.int32)
    i = jnp.int32(0x5F3759DF) - (i >> 1)
    y = lax.bitcast_convert_type(i, jnp.float32)
    for _ in range(3):
        y = y * (1.5 - 0.5 * x * y * y)
    return y


def _sc_dirs(pxyz, ei_i, ei_j, sxyz, edge_attr, n_nodes, n_edges):
    d = edge_attr.shape[1]
    total_chunks = n_edges // _C
    slots = (total_chunks + _NW - 1) // _NW
    sub_rows = 128
    n_sub = _C // sub_rows
    mesh = plsc.VectorSubcoreMesh(core_axis_name="c", subcore_axis_name="s")

    @functools.partial(
        pl.kernel,
        mesh=mesh,
        out_type=[jax.ShapeDtypeStruct((n_edges,), jnp.float32)] * 3
        + [jax.ShapeDtypeStruct((n_edges * d,), jnp.float32)],
        scratch_types=[
            pltpu.VMEM((sub_rows, d), jnp.float32),
            pltpu.VMEM((_C * d,), jnp.float32),
            pltpu.VMEM_SHARED((n_nodes,), jnp.float32),
            pltpu.VMEM_SHARED((n_nodes,), jnp.float32),
            pltpu.VMEM_SHARED((n_nodes,), jnp.float32),
            pltpu.VMEM((_C,), jnp.int32),
            pltpu.VMEM((_C,), jnp.int32),
            pltpu.VMEM((_C,), jnp.float32),
            pltpu.VMEM((_C,), jnp.float32),
            pltpu.VMEM((_C,), jnp.float32),
            pltpu.VMEM((_C,), jnp.float32),
            pltpu.VMEM((_C,), jnp.float32),
            pltpu.VMEM((_C,), jnp.float32),
            pltpu.VMEM((_C,), jnp.float32),
            pltpu.VMEM((_C,), jnp.float32),
            pltpu.VMEM((_C,), jnp.float32),
            pltpu.VMEM((_C,), jnp.float32),
            pltpu.VMEM((_C,), jnp.float32),
            pltpu.VMEM((_C,), jnp.float32),
            pltpu.SemaphoreType.DMA,
            pltpu.SemaphoreType.DMA,
        ],
    )
    def k(px_hbm, py_hbm, pz_hbm, ii_hbm, jj_hbm,
          shx_hbm, shy_hbm, shz_hbm, ea_hbm,
          oux, ouy, ouz, oxp,
          ea_v, xp_v,
          px_sp, py_sp, pz_sp,
          ii_v, jj_v, sx_v, sy_v, sz_v,
          pxi_v, pyi_v, pzi_v, pxj_v, pyj_v, pzj_v,
          ux_v, uy_v, uz_v, semi, semj):
        cid = lax.axis_index("c")
        sid = lax.axis_index("s")
        wid = sid * _NC + cid

        @pl.when(sid == 0)
        def _():
            pltpu.sync_copy(px_hbm, px_sp)
            pltpu.sync_copy(py_hbm, py_sp)
            pltpu.sync_copy(pz_hbm, pz_sp)

        plsc.subcore_barrier()

        def chunk_body(t, carry):
            chunk = wid + t * _NW

            @pl.when(chunk < total_chunks)
            def _():
                base = pl.multiple_of(chunk * _C, _C)
                pltpu.sync_copy(ii_hbm.at[pl.ds(base, _C)], ii_v)
                pltpu.sync_copy(jj_hbm.at[pl.ds(base, _C)], jj_v)
                cpi1 = pltpu.async_copy(px_sp.at[ii_v], pxi_v, semi)
                cpi2 = pltpu.async_copy(py_sp.at[ii_v], pyi_v, semi)
                cpi3 = pltpu.async_copy(pz_sp.at[ii_v], pzi_v, semi)
                cpj1 = pltpu.async_copy(px_sp.at[jj_v], pxj_v, semj)
                cpj2 = pltpu.async_copy(py_sp.at[jj_v], pyj_v, semj)
                cpj3 = pltpu.async_copy(pz_sp.at[jj_v], pzj_v, semj)
                pltpu.sync_copy(shx_hbm.at[pl.ds(base, _C)], sx_v)
                pltpu.sync_copy(shy_hbm.at[pl.ds(base, _C)], sy_v)
                pltpu.sync_copy(shz_hbm.at[pl.ds(base, _C)], sz_v)
                cpi1.wait()
                cpi2.wait()
                cpi3.wait()
                cpj1.wait()
                cpj2.wait()
                cpj3.wait()

                def group_body(gi, gcarry):
                    o = pl.ds(pl.multiple_of(gi * 16, 16), 16)
                    dx = pxi_v[o] + sx_v[o] - pxj_v[o]
                    dy = pyi_v[o] + sy_v[o] - pyj_v[o]
                    dz = pzi_v[o] + sz_v[o] - pzj_v[o]
                    r2 = dx * dx + dy * dy + dz * dz
                    rinv = _newton_rsqrt(r2)
                    ux_v[o] = dx * rinv
                    uy_v[o] = dy * rinv
                    uz_v[o] = dz * rinv
                    return gcarry

                lax.fori_loop(0, _G, group_body, 0)
                pltpu.sync_copy(ux_v, oux.at[pl.ds(base, _C)])
                pltpu.sync_copy(uy_v, ouy.at[pl.ds(base, _C)])
                pltpu.sync_copy(uz_v, ouz.at[pl.ds(base, _C)])

                def sub_body(sb, scarry):
                    off = pl.multiple_of(base + sb * sub_rows, sub_rows)
                    pltpu.sync_copy(ea_hbm.at[pl.ds(off, sub_rows), :], ea_v)
                    xb = pl.multiple_of(sb * sub_rows * d, sub_rows * d)
                    for r in range(sub_rows):
                        xp_v[pl.ds(xb + r * d, d)] = ea_v[r]
                    return scarry

                lax.fori_loop(0, n_sub, sub_body, 0)
                pltpu.sync_copy(xp_v, oxp.at[pl.ds(base * d, _C * d)])

            return carry

        lax.fori_loop(0, slots, chunk_body, 0)

    oux_, ouy_, ouz_, oxp_ = k(pxyz[0], pxyz[1], pxyz[2], ei_i, ei_j,
                               sxyz[0], sxyz[1], sxyz[2], edge_attr)
    return (oux_, ouy_, ouz_), oxp_


def _sc_scatter(ei_i, s_flat, u3, zeros_n, n_nodes, n_edges):
    total_chunks = n_edges // _C
    slots = (total_chunks + _NW - 1) // _NW
    mesh = plsc.VectorSubcoreMesh(core_axis_name="c", subcore_axis_name="s")

    @functools.partial(
        pl.kernel,
        mesh=mesh,
        out_type=[jax.ShapeDtypeStruct((n_nodes,), jnp.float32)] * 6,
        scratch_types=[
            pltpu.VMEM_SHARED((n_nodes,), jnp.float32),
            pltpu.VMEM_SHARED((n_nodes,), jnp.float32),
            pltpu.VMEM_SHARED((n_nodes,), jnp.float32),
            pltpu.VMEM((_C,), jnp.int32),
            pltpu.VMEM((_C,), jnp.float32),
            pltpu.VMEM((_C,), jnp.float32),
            pltpu.VMEM((_C,), jnp.float32),
            pltpu.VMEM((_C,), jnp.float32),
            pltpu.VMEM((_C,), jnp.float32),
            pltpu.VMEM((_C,), jnp.float32),
            pltpu.VMEM((_C,), jnp.float32),
        ],
    )
    def k(ii_hbm, s_hbm, ux_hbm, uy_hbm, uz_hbm, zero_hbm,
          ox0, oy0, oz0, ox1, oy1, oz1,
          fx_sp, fy_sp, fz_sp,
          ii_v, s_v, ux_v, uy_v, uz_v, fx_v, fy_v, fz_v):
        cid = lax.axis_index("c")
        sid = lax.axis_index("s")
        wid = sid * _NC + cid

        @pl.when(sid == 0)
        def _():
            pltpu.sync_copy(zero_hbm, fx_sp)
            pltpu.sync_copy(zero_hbm, fy_sp)
            pltpu.sync_copy(zero_hbm, fz_sp)

        plsc.subcore_barrier()

        def chunk_body(t, carry):
            chunk = wid + t * _NW

            @pl.when(chunk < total_chunks)
            def _():
                base = pl.multiple_of(chunk * _C, _C)
                pltpu.sync_copy(ii_hbm.at[pl.ds(base, _C)], ii_v)
                pltpu.sync_copy(s_hbm.at[pl.ds(base, _C)], s_v)
                pltpu.sync_copy(ux_hbm.at[pl.ds(base, _C)], ux_v)
                pltpu.sync_copy(uy_hbm.at[pl.ds(base, _C)], uy_v)
                pltpu.sync_copy(uz_hbm.at[pl.ds(base, _C)], uz_v)

                def group_body(gi, gcarry):
                    o = pl.ds(pl.multiple_of(gi * 16, 16), 16)
                    f = s_v[o]
                    fx_v[o] = ux_v[o] * f
                    fy_v[o] = uy_v[o] * f
                    fz_v[o] = uz_v[o] * f
                    return gcarry

                lax.fori_loop(0, _G, group_body, 0)
                pltpu.sync_copy(fx_v, fx_sp.at[ii_v], add=True)
                pltpu.sync_copy(fy_v, fy_sp.at[ii_v], add=True)
                pltpu.sync_copy(fz_v, fz_sp.at[ii_v], add=True)

            return carry

        lax.fori_loop(0, slots, chunk_body, 0)
        plsc.subcore_barrier()

        @pl.when((sid == 0) & (cid == 0))
        def _():
            pltpu.sync_copy(fx_sp, ox0)
            pltpu.sync_copy(fy_sp, oy0)
            pltpu.sync_copy(fz_sp, oz0)

        @pl.when((sid == 0) & (cid == 1))
        def _():
            pltpu.sync_copy(fx_sp, ox1)
            pltpu.sync_copy(fy_sp, oy1)
            pltpu.sync_copy(fz_sp, oz1)

    return k(ei_i, s_flat, u3[0], u3[1], u3[2], zeros_n)


def kernel(pos, edge_index, nbr_shift, edge_attr,
           W0, b0, g0, be0, W1, b1, g1, be1, W2, b2, g2, be2, Wo, bo):
    n_nodes = pos.shape[0]
    n_edges = edge_attr.shape[0]
    d = edge_attr.shape[1]
    rows = n_edges // _PACK
    grid = rows // _BE

    f32 = jnp.float32
    eye8 = jnp.eye(_PACK, dtype=f32)

    wbs = [jnp.kron(eye8, W) for W in (W0, W1, W2)]
    bs = (b0, b1, b2)
    gs = (g0, g1, g2)
    bes = (be0, be1, be2)

    ei_j = edge_index[0]
    ei_i = edge_index[1]
    sxyz = tuple(nbr_shift[:, k] for k in range(3))
    pxyz = tuple(pos[:, k] for k in range(3))
    zeros_n = jnp.zeros((n_nodes,), f32)

    u3, xp_flat = _sc_dirs(pxyz, ei_i, ei_j, sxyz, edge_attr, n_nodes, n_edges)
    xp = xp_flat.reshape(rows, _PACK * d)

    count = jnp.float32(n_edges)
    known = []
    for l in range(3):
        sum_v, sq_v = _stats_pass(xp, known, wbs[l], grid)
        a, c2 = _fold_stats(sum_v, sq_v, count, bs[l], gs[l], bes[l])
        known.append((wbs[l], a, c2))

    wo_sel = jnp.kron(eye8, Wo)
    bo_t = jnp.broadcast_to(bo, (128,))[None, :]
    s2d = _final_pass(xp, known, wo_sel, bo_t, grid, rows)
    s_flat = s2d.T.reshape(-1)

    fx0, fy0, fz0, fx1, fy1, fz1 = _sc_scatter(
        ei_i, s_flat, u3, zeros_n, n_nodes, n_edges)
    return jnp.stack([fx0 + fx1, fy0 + fy1, fz0 + fz1], axis=1)


# speedup vs baseline: 1.5216x; 1.5216x over previous
import functools

import jax
import jax.numpy as jnp
from jax import lax
from jax.experimental import pallas as pl
from jax.experimental.pallas import tpu as pltpu
from jax.experimental.pallas import tpu_sc as plsc

_EPS = 1e-5
_PACK = 8
_BE = 3200

_NC = 2
_NS = 16
_NW = _NC * _NS
_C = 1600
_G = _C // 16


_LOG2E = 1.4426950408889634
_LN2 = 0.6931471805599453


def _softplus(z):
    return jnp.maximum(z, 0.0) + _LN2 * jnp.log2(1.0 + jnp.exp2(jnp.abs(z) * -_LOG2E))


def _make_stats_body(nl):
    def body(*args):
        x_ref = args[0]
        p = 1
        h = x_ref[...]
        for _ in range(nl):
            wb = args[p][...]
            a = args[p + 1][...]
            c2 = args[p + 2][...]
            p += 3
            h = _softplus(jnp.dot(h, wb, preferred_element_type=jnp.float32) * a + c2)
        wb = args[p][...]
        sum_ref = args[p + 1]
        sq_ref = args[p + 2]
        h = jnp.dot(h, wb, preferred_element_type=jnp.float32)

        @pl.when(pl.program_id(0) == 0)
        def _():
            sum_ref[...] = jnp.zeros_like(sum_ref)
            sq_ref[...] = jnp.zeros_like(sq_ref)

        sum_ref[...] += jnp.sum(h, axis=0, keepdims=True)
        sq_ref[...] += jnp.sum(h * h, axis=0, keepdims=True)

    return body


def _make_final_body(nl):
    def body(*args):
        x_ref = args[0]
        p = 1
        h = x_ref[...]
        for _ in range(nl):
            wb = args[p][...]
            a = args[p + 1][...]
            c2 = args[p + 2][...]
            p += 3
            h = _softplus(jnp.dot(h, wb, preferred_element_type=jnp.float32) * a + c2)
        wo = args[p][...]
        bo = args[p + 1]
        out_ref = args[p + 2]
        s8 = jnp.dot(h, wo, preferred_element_type=jnp.float32) + bo[0, 0]
        out_ref[...] = s8.T

    return body


def _full_spec():
    return pl.BlockSpec((128, 128), lambda i: (0, 0))


def _vec_spec():
    return pl.BlockSpec((1, 128), lambda i: (0, 0))


def _stats_pass(xp, known, wb_next, grid):
    ops = [xp]
    specs = [pl.BlockSpec((_BE, 128), lambda i: (i, 0))]
    for (wb, a, c2) in known:
        ops += [wb, a, c2]
        specs += [_full_spec(), _vec_spec(), _vec_spec()]
    ops += [wb_next]
    specs += [_full_spec()]
    return pl.pallas_call(
        _make_stats_body(len(known)),
        grid=(grid,),
        in_specs=specs,
        out_specs=[_vec_spec(), _vec_spec()],
        out_shape=[jax.ShapeDtypeStruct((1, 128), jnp.float32)] * 2,
    )(*ops)


def _final_pass(xp, known, wo_sel, bo_t, grid, rows):
    ops = [xp]
    specs = [pl.BlockSpec((_BE, 128), lambda i: (i, 0))]
    for (wb, a, c2) in known:
        ops += [wb, a, c2]
        specs += [_full_spec(), _vec_spec(), _vec_spec()]
    ops += [wo_sel, bo_t]
    specs += [pl.BlockSpec((128, _PACK), lambda i: (0, 0)), _vec_spec()]
    return pl.pallas_call(
        _make_final_body(len(known)),
        grid=(grid,),
        in_specs=specs,
        out_specs=pl.BlockSpec((_PACK, _BE), lambda i: (0, i)),
        out_shape=jax.ShapeDtypeStruct((_PACK, rows), jnp.float32),
    )(*ops)


def _fold_stats(sum_v, sq_v, count, b, g, be):
    m_raw = sum_v.reshape(_PACK, 16).sum(axis=0) / count
    q_raw = sq_v.reshape(_PACK, 16).sum(axis=0) / count
    mean = m_raw + b
    var = q_raw + 2.0 * b * m_raw + b * b - mean * mean
    a16 = g / jnp.sqrt(var + _EPS)
    c16 = be - mean * a16
    c2 = b * a16 + c16
    return jnp.tile(a16, _PACK)[None, :], jnp.tile(c2, _PACK)[None, :]


def _newton_rsqrt(x):
    i = lax.bitcast_convert_type(x, jnp.int32)
    i = jnp.int32(0x5F3759DF) - (i >> 1)
    y = lax.bitcast_convert_type(i, jnp.float32)
    for _ in range(3):
        y = y * (1.5 - 0.5 * x * y * y)
    return y


def _sc_forces(pxyz, ei_i, ei_j, sxyz, s_flat, zeros_n, n_nodes, n_edges):
    total_chunks = n_edges // _C
    slots = (total_chunks + _NW - 1) // _NW
    mesh = plsc.VectorSubcoreMesh(core_axis_name="c", subcore_axis_name="s")

    @functools.partial(
        pl.kernel,
        mesh=mesh,
        out_type=[jax.ShapeDtypeStruct((n_nodes,), jnp.float32)] * 6,
        scratch_types=[
            pltpu.VMEM_SHARED((n_nodes,), jnp.float32),
            pltpu.VMEM_SHARED((n_nodes,), jnp.float32),
            pltpu.VMEM_SHARED((n_nodes,), jnp.float32),
            pltpu.VMEM_SHARED((n_nodes,), jnp.float32),
            pltpu.VMEM_SHARED((n_nodes,), jnp.float32),
            pltpu.VMEM_SHARED((n_nodes,), jnp.float32),
            pltpu.VMEM((_C,), jnp.int32),
            pltpu.VMEM((_C,), jnp.int32),
            pltpu.VMEM((_C,), jnp.float32),
            pltpu.VMEM((_C,), jnp.float32),
            pltpu.VMEM((_C,), jnp.float32),
            pltpu.VMEM((_C,), jnp.float32),
            pltpu.VMEM((_C,), jnp.float32),
            pltpu.VMEM((_C,), jnp.float32),
            pltpu.VMEM((_C,), jnp.float32),
            pltpu.VMEM((_C,), jnp.float32),
            pltpu.VMEM((_C,), jnp.float32),
            pltpu.VMEM((_C,), jnp.float32),
            pltpu.VMEM((_C,), jnp.float32),
            pltpu.VMEM((_C,), jnp.float32),
            pltpu.VMEM((_C,), jnp.float32),
            pltpu.SemaphoreType.DMA,
            pltpu.SemaphoreType.DMA,
        ],
    )
    def k(px_hbm, py_hbm, pz_hbm, ii_hbm, jj_hbm,
          shx_hbm, shy_hbm, shz_hbm, s_hbm, zero_hbm,
          ox0, oy0, oz0, ox1, oy1, oz1,
          px_sp, py_sp, pz_sp, fx_sp, fy_sp, fz_sp,
          ii_v, jj_v, sx_v, sy_v, sz_v, s_v,
          pxi_v, pyi_v, pzi_v, pxj_v, pyj_v, pzj_v,
          fx_v, fy_v, fz_v, semi, semj):
        cid = lax.axis_index("c")
        sid = lax.axis_index("s")
        wid = sid * _NC + cid

        @pl.when(sid == 0)
        def _():
            pltpu.sync_copy(px_hbm, px_sp)
            pltpu.sync_copy(py_hbm, py_sp)
            pltpu.sync_copy(pz_hbm, pz_sp)
            pltpu.sync_copy(zero_hbm, fx_sp)
            pltpu.sync_copy(zero_hbm, fy_sp)
            pltpu.sync_copy(zero_hbm, fz_sp)

        plsc.subcore_barrier()

        def chunk_body(t, carry):
            chunk = wid + t * _NW

            @pl.when(chunk < total_chunks)
            def _():
                base = pl.multiple_of(chunk * _C, _C)
                pltpu.sync_copy(ii_hbm.at[pl.ds(base, _C)], ii_v)
                pltpu.sync_copy(jj_hbm.at[pl.ds(base, _C)], jj_v)
                cpi1 = pltpu.async_copy(px_sp.at[ii_v], pxi_v, semi)
                cpi2 = pltpu.async_copy(py_sp.at[ii_v], pyi_v, semi)
                cpi3 = pltpu.async_copy(pz_sp.at[ii_v], pzi_v, semi)
                cpj1 = pltpu.async_copy(px_sp.at[jj_v], pxj_v, semj)
                cpj2 = pltpu.async_copy(py_sp.at[jj_v], pyj_v, semj)
                cpj3 = pltpu.async_copy(pz_sp.at[jj_v], pzj_v, semj)
                pltpu.sync_copy(shx_hbm.at[pl.ds(base, _C)], sx_v)
                pltpu.sync_copy(shy_hbm.at[pl.ds(base, _C)], sy_v)
                pltpu.sync_copy(shz_hbm.at[pl.ds(base, _C)], sz_v)
                pltpu.sync_copy(s_hbm.at[pl.ds(base, _C)], s_v)
                cpi1.wait()
                cpi2.wait()
                cpi3.wait()
                cpj1.wait()
                cpj2.wait()
                cpj3.wait()

                def group_body(gi, gcarry):
                    o = pl.ds(pl.multiple_of(gi * 16, 16), 16)
                    dx = pxi_v[o] + sx_v[o] - pxj_v[o]
                    dy = pyi_v[o] + sy_v[o] - pyj_v[o]
                    dz = pzi_v[o] + sz_v[o] - pzj_v[o]
                    r2 = dx * dx + dy * dy + dz * dz
                    f = s_v[o] * _newton_rsqrt(r2)
                    fx_v[o] = dx * f
                    fy_v[o] = dy * f
                    fz_v[o] = dz * f
                    return gcarry

                lax.fori_loop(0, _G, group_body, 0)
                pltpu.sync_copy(fx_v, fx_sp.at[ii_v], add=True)
                pltpu.sync_copy(fy_v, fy_sp.at[ii_v], add=True)
                pltpu.sync_copy(fz_v, fz_sp.at[ii_v], add=True)

            return carry

        lax.fori_loop(0, slots, chunk_body, 0)
        plsc.subcore_barrier()

        @pl.when((sid == 0) & (cid == 0))
        def _():
            pltpu.sync_copy(fx_sp, ox0)
            pltpu.sync_copy(fy_sp, oy0)
            pltpu.sync_copy(fz_sp, oz0)

        @pl.when((sid == 0) & (cid == 1))
        def _():
            pltpu.sync_copy(fx_sp, ox1)
            pltpu.sync_copy(fy_sp, oy1)
            pltpu.sync_copy(fz_sp, oz1)

    return k(pxyz[0], pxyz[1], pxyz[2], ei_i, ei_j,
             sxyz[0], sxyz[1], sxyz[2], s_flat, zeros_n)


def kernel(pos, edge_index, nbr_shift, edge_attr,
           W0, b0, g0, be0, W1, b1, g1, be1, W2, b2, g2, be2, Wo, bo):
    n_nodes = pos.shape[0]
    n_edges = edge_attr.shape[0]
    d = edge_attr.shape[1]
    rows = n_edges // _PACK
    grid = rows // _BE

    f32 = jnp.float32
    eye8 = jnp.eye(_PACK, dtype=f32)

    wbs = [jnp.kron(eye8, W) for W in (W0, W1, W2)]
    bs = (b0, b1, b2)
    gs = (g0, g1, g2)
    bes = (be0, be1, be2)

    ei_j = edge_index[0]
    ei_i = edge_index[1]
    sxyz = tuple(nbr_shift[:, k] for k in range(3))
    pxyz = tuple(pos[:, k] for k in range(3))
    zeros_n = jnp.zeros((n_nodes,), f32)

    xp = edge_attr.reshape(rows, _PACK * d)

    count = jnp.float32(n_edges)
    known = []
    for l in range(3):
        sum_v, sq_v = _stats_pass(xp, known, wbs[l], grid)
        a, c2 = _fold_stats(sum_v, sq_v, count, bs[l], gs[l], bes[l])
        known.append((wbs[l], a, c2))

    wo_sel = jnp.kron(eye8, Wo)
    bo_t = jnp.broadcast_to(bo, (128,))[None, :]
    s2d = _final_pass(xp, known, wo_sel, bo_t, grid, rows)
    s_flat = s2d.T.reshape(-1)

    fx0, fy0, fz0, fx1, fy1, fz1 = _sc_forces(
        pxyz, ei_i, ei_j, sxyz, s_flat, zeros_n, n_nodes, n_edges)
    return jnp.stack([fx0 + fx1, fy0 + fy1, fz0 + fz1], axis=1)


# speedup vs baseline: 1.6455x; 1.0814x over previous
import functools

import jax
import jax.numpy as jnp
from jax import lax
from jax.experimental import pallas as pl
from jax.experimental.pallas import tpu as pltpu
from jax.experimental.pallas import tpu_sc as plsc

_EPS = 1e-5
_PACK = 8
_BE = 16000

_NC = 2
_NS = 16
_NW = _NC * _NS
_C = 3200
_G = _C // 16


_LOG2E = 1.4426950408889634
_LN2 = 0.6931471805599453


def _softplus(z):
    return jnp.maximum(z, 0.0) + _LN2 * jnp.log2(1.0 + jnp.exp2(jnp.abs(z) * -_LOG2E))


def _make_stats_body(nl):
    def body(*args):
        x_ref = args[0]
        p = 1
        h = x_ref[...]
        for _ in range(nl):
            wb = args[p][...]
            a = args[p + 1][...]
            c2 = args[p + 2][...]
            p += 3
            h = _softplus(jnp.dot(h, wb, preferred_element_type=jnp.float32) * a + c2)
        wb = args[p][...]
        sum_ref = args[p + 1]
        sq_ref = args[p + 2]
        h = jnp.dot(h, wb, preferred_element_type=jnp.float32)

        @pl.when(pl.program_id(0) == 0)
        def _():
            sum_ref[...] = jnp.zeros_like(sum_ref)
            sq_ref[...] = jnp.zeros_like(sq_ref)

        sum_ref[...] += jnp.sum(h, axis=0, keepdims=True)
        sq_ref[...] += jnp.sum(h * h, axis=0, keepdims=True)

    return body


def _make_final_body(nl):
    def body(*args):
        x_ref = args[0]
        p = 1
        h = x_ref[...]
        for _ in range(nl):
            wb = args[p][...]
            a = args[p + 1][...]
            c2 = args[p + 2][...]
            p += 3
            h = _softplus(jnp.dot(h, wb, preferred_element_type=jnp.float32) * a + c2)
        wo = args[p][...]
        bo = args[p + 1]
        out_ref = args[p + 2]
        s8 = jnp.dot(h, wo, preferred_element_type=jnp.float32) + bo[0, 0]
        out_ref[...] = s8.T

    return body


def _full_spec():
    return pl.BlockSpec((128, 128), lambda i: (0, 0))


def _vec_spec():
    return pl.BlockSpec((1, 128), lambda i: (0, 0))


def _stats_pass(xp, known, wb_next, grid):
    ops = [xp]
    specs = [pl.BlockSpec((_BE, 128), lambda i: (i, 0))]
    for (wb, a, c2) in known:
        ops += [wb, a, c2]
        specs += [_full_spec(), _vec_spec(), _vec_spec()]
    ops += [wb_next]
    specs += [_full_spec()]
    return pl.pallas_call(
        _make_stats_body(len(known)),
        grid=(grid,),
        in_specs=specs,
        out_specs=[_vec_spec(), _vec_spec()],
        out_shape=[jax.ShapeDtypeStruct((1, 128), jnp.float32)] * 2,
    )(*ops)


def _final_pass(xp, known, wo_sel, bo_t, grid, rows):
    ops = [xp]
    specs = [pl.BlockSpec((_BE, 128), lambda i: (i, 0))]
    for (wb, a, c2) in known:
        ops += [wb, a, c2]
        specs += [_full_spec(), _vec_spec(), _vec_spec()]
    ops += [wo_sel, bo_t]
    specs += [pl.BlockSpec((128, _PACK), lambda i: (0, 0)), _vec_spec()]
    return pl.pallas_call(
        _make_final_body(len(known)),
        grid=(grid,),
        in_specs=specs,
        out_specs=pl.BlockSpec((_PACK, _BE), lambda i: (0, i)),
        out_shape=jax.ShapeDtypeStruct((_PACK, rows), jnp.float32),
    )(*ops)


def _fold_stats(sum_v, sq_v, count, b, g, be):
    m_raw = sum_v.reshape(_PACK, 16).sum(axis=0) / count
    q_raw = sq_v.reshape(_PACK, 16).sum(axis=0) / count
    mean = m_raw + b
    var = q_raw + 2.0 * b * m_raw + b * b - mean * mean
    a16 = g / jnp.sqrt(var + _EPS)
    c16 = be - mean * a16
    c2 = b * a16 + c16
    return jnp.tile(a16, _PACK)[None, :], jnp.tile(c2, _PACK)[None, :]


def _newton_rsqrt(x):
    i = lax.bitcast_convert_type(x, jnp.int32)
    i = jnp.int32(0x5F3759DF) - (i >> 1)
    y = lax.bitcast_convert_type(i, jnp.float32)
    for _ in range(3):
        y = y * (1.5 - 0.5 * x * y * y)
    return y


def _sc_forces(pxyz, ei_i, ei_j, sxyz, s_flat, zeros_n, n_nodes, n_edges):
    total_chunks = n_edges // _C
    slots = (total_chunks + _NW - 1) // _NW
    mesh = plsc.VectorSubcoreMesh(core_axis_name="c", subcore_axis_name="s")

    @functools.partial(
        pl.kernel,
        mesh=mesh,
        out_type=[jax.ShapeDtypeStruct((n_nodes,), jnp.float32)] * 6,
        scratch_types=[
            pltpu.VMEM_SHARED((n_nodes,), jnp.float32),
            pltpu.VMEM_SHARED((n_nodes,), jnp.float32),
            pltpu.VMEM_SHARED((n_nodes,), jnp.float32),
            pltpu.VMEM_SHARED((n_nodes,), jnp.float32),
            pltpu.VMEM_SHARED((n_nodes,), jnp.float32),
            pltpu.VMEM_SHARED((n_nodes,), jnp.float32),
            pltpu.VMEM((_C,), jnp.int32),
            pltpu.VMEM((_C,), jnp.int32),
            pltpu.VMEM((_C,), jnp.float32),
            pltpu.VMEM((_C,), jnp.float32),
            pltpu.VMEM((_C,), jnp.float32),
            pltpu.VMEM((_C,), jnp.float32),
            pltpu.VMEM((_C,), jnp.float32),
            pltpu.VMEM((_C,), jnp.float32),
            pltpu.VMEM((_C,), jnp.float32),
            pltpu.VMEM((_C,), jnp.float32),
            pltpu.VMEM((_C,), jnp.float32),
            pltpu.VMEM((_C,), jnp.float32),
            pltpu.VMEM((_C,), jnp.float32),
            pltpu.VMEM((_C,), jnp.float32),
            pltpu.VMEM((_C,), jnp.float32),
            pltpu.SemaphoreType.DMA,
            pltpu.SemaphoreType.DMA,
        ],
    )
    def k(px_hbm, py_hbm, pz_hbm, ii_hbm, jj_hbm,
          shx_hbm, shy_hbm, shz_hbm, s_hbm, zero_hbm,
          ox0, oy0, oz0, ox1, oy1, oz1,
          px_sp, py_sp, pz_sp, fx_sp, fy_sp, fz_sp,
          ii_v, jj_v, sx_v, sy_v, sz_v, s_v,
          pxi_v, pyi_v, pzi_v, pxj_v, pyj_v, pzj_v,
          fx_v, fy_v, fz_v, semi, semj):
        cid = lax.axis_index("c")
        sid = lax.axis_index("s")
        wid = sid * _NC + cid

        @pl.when(sid == 0)
        def _():
            pltpu.sync_copy(px_hbm, px_sp)
            pltpu.sync_copy(py_hbm, py_sp)
            pltpu.sync_copy(pz_hbm, pz_sp)
            pltpu.sync_copy(zero_hbm, fx_sp)
            pltpu.sync_copy(zero_hbm, fy_sp)
            pltpu.sync_copy(zero_hbm, fz_sp)

        plsc.subcore_barrier()

        def chunk_body(t, carry):
            chunk = wid + t * _NW

            @pl.when(chunk < total_chunks)
            def _():
                base = pl.multiple_of(chunk * _C, _C)
                pltpu.sync_copy(ii_hbm.at[pl.ds(base, _C)], ii_v)
                pltpu.sync_copy(jj_hbm.at[pl.ds(base, _C)], jj_v)
                cpi1 = pltpu.async_copy(px_sp.at[ii_v], pxi_v, semi)
                cpi2 = pltpu.async_copy(py_sp.at[ii_v], pyi_v, semi)
                cpi3 = pltpu.async_copy(pz_sp.at[ii_v], pzi_v, semi)
                cpj1 = pltpu.async_copy(px_sp.at[jj_v], pxj_v, semj)
                cpj2 = pltpu.async_copy(py_sp.at[jj_v], pyj_v, semj)
                cpj3 = pltpu.async_copy(pz_sp.at[jj_v], pzj_v, semj)
                pltpu.sync_copy(shx_hbm.at[pl.ds(base, _C)], sx_v)
                pltpu.sync_copy(shy_hbm.at[pl.ds(base, _C)], sy_v)
                pltpu.sync_copy(shz_hbm.at[pl.ds(base, _C)], sz_v)
                pltpu.sync_copy(s_hbm.at[pl.ds(base, _C)], s_v)
                cpi1.wait()
                cpi2.wait()
                cpi3.wait()
                cpj1.wait()
                cpj2.wait()
                cpj3.wait()

                def group_body(gi, gcarry):
                    o = pl.ds(pl.multiple_of(gi * 16, 16), 16)
                    dx = pxi_v[o] + sx_v[o] - pxj_v[o]
                    dy = pyi_v[o] + sy_v[o] - pyj_v[o]
                    dz = pzi_v[o] + sz_v[o] - pzj_v[o]
                    r2 = dx * dx + dy * dy + dz * dz
                    f = s_v[o] * _newton_rsqrt(r2)
                    fx_v[o] = dx * f
                    fy_v[o] = dy * f
                    fz_v[o] = dz * f
                    return gcarry

                lax.fori_loop(0, _G, group_body, 0)
                pltpu.sync_copy(fx_v, fx_sp.at[ii_v], add=True)
                pltpu.sync_copy(fy_v, fy_sp.at[ii_v], add=True)
                pltpu.sync_copy(fz_v, fz_sp.at[ii_v], add=True)

            return carry

        lax.fori_loop(0, slots, chunk_body, 0)
        plsc.subcore_barrier()

        @pl.when((sid == 0) & (cid == 0))
        def _():
            pltpu.sync_copy(fx_sp, ox0)
            pltpu.sync_copy(fy_sp, oy0)
            pltpu.sync_copy(fz_sp, oz0)

        @pl.when((sid == 0) & (cid == 1))
        def _():
            pltpu.sync_copy(fx_sp, ox1)
            pltpu.sync_copy(fy_sp, oy1)
            pltpu.sync_copy(fz_sp, oz1)

    return k(pxyz[0], pxyz[1], pxyz[2], ei_i, ei_j,
             sxyz[0], sxyz[1], sxyz[2], s_flat, zeros_n)


def kernel(pos, edge_index, nbr_shift, edge_attr,
           W0, b0, g0, be0, W1, b1, g1, be1, W2, b2, g2, be2, Wo, bo):
    n_nodes = pos.shape[0]
    n_edges = edge_attr.shape[0]
    d = edge_attr.shape[1]
    rows = n_edges // _PACK
    grid = rows // _BE

    f32 = jnp.float32
    eye8 = jnp.eye(_PACK, dtype=f32)

    wbs = [jnp.kron(eye8, W) for W in (W0, W1, W2)]
    bs = (b0, b1, b2)
    gs = (g0, g1, g2)
    bes = (be0, be1, be2)

    ei_j = edge_index[0]
    ei_i = edge_index[1]
    sxyz = tuple(nbr_shift[:, k] for k in range(3))
    pxyz = tuple(pos[:, k] for k in range(3))
    zeros_n = jnp.zeros((n_nodes,), f32)

    xp = edge_attr.reshape(rows, _PACK * d)

    count = jnp.float32(n_edges)
    known = []
    for l in range(3):
        sum_v, sq_v = _stats_pass(xp, known, wbs[l], grid)
        a, c2 = _fold_stats(sum_v, sq_v, count, bs[l], gs[l], bes[l])
        known.append((wbs[l], a, c2))

    wo_sel = jnp.kron(eye8, Wo)
    bo_t = jnp.broadcast_to(bo, (128,))[None, :]
    s2d = _final_pass(xp, known, wo_sel, bo_t, grid, rows)
    s_flat = s2d.T.reshape(-1)

    fx0, fy0, fz0, fx1, fy1, fz1 = _sc_forces(
        pxyz, ei_i, ei_j, sxyz, s_flat, zeros_n, n_nodes, n_edges)
    return jnp.stack([fx0 + fx1, fy0 + fy1, fz0 + fz1], axis=1)
